# baseline (speedup 1.0000x reference)
import functools

import jax
import jax.numpy as jnp
from jax import lax
from jax.experimental import pallas as pl
from jax.experimental.pallas import tpu as pltpu
from jax.experimental.pallas import tpu_sc as plsc

B = 4096
L = 200
V = 100000
D = 128

NC = 2
NS = 16
LANES = 16
NW = NC * NS
BPW = B // NW
CHUNKS = ((0, 104), (104, 96))
CHMAX = 104
NBUF = 6
NVREG = D // LANES


def _bow_body(idx_hbm, table_hbm, out_hbm, idx_v, buf_v, out_v,
              sem0, sem1, sem2, sem3, sem4, sem5):
    wid = lax.axis_index("s") * NC + lax.axis_index("c")
    sems = (sem0, sem1, sem2, sem3, sem4, sem5)
    inv = jnp.full((LANES,), 1.0 / L, dtype=jnp.float32)

    pltpu.sync_copy(idx_hbm.at[pl.ds(wid * BPW, BPW)], idx_v)

    def start_gather(slot, bag, ci):
        off, ch = CHUNKS[ci]
        pltpu.make_async_copy(
            table_hbm.at[idx_v.at[bag, pl.ds(off, ch)]],
            buf_v.at[slot, pl.ds(0, ch)],
            sems[slot],
        ).start()

    def drain(slot, ci):
        off, ch = CHUNKS[ci]
        pltpu.make_async_copy(
            table_hbm.at[idx_v.at[0, pl.ds(off, ch)]],
            buf_v.at[slot, pl.ds(0, ch)],
            sems[slot],
        ).wait()

    UNROLL = 8

    def consume(slot, ci, accs):
        _, ch = CHUNKS[ci]

        def row_add(i, accs):
            l = i * UNROLL
            out = []
            for k in range(NVREG):
                a = accs[k]
                parts = [
                    buf_v[slot, l + u, pl.ds(k * LANES, LANES)]
                    for u in range(UNROLL)
                ]
                while len(parts) > 1:
                    parts = [
                        parts[j] + parts[j + 1]
                        for j in range(0, len(parts), 2)
                    ]
                out.append(a + parts[0])
            return tuple(out)

        return lax.fori_loop(0, ch // UNROLL, row_add, accs)

    def store(bag, accs):
        for k in range(NVREG):
            out_v[bag, pl.ds(k * LANES, LANES)] = accs[k] * inv

    GRP = NBUF // 2
    NSTEPS = 41

    for bagoff in range(GRP):
        for ci in range(2):
            start_gather(2 * bagoff + ci, bagoff, ci)

    def step(i, _):
        for bagoff in range(GRP):
            bag = GRP * i + bagoff
            accs = tuple(
                jnp.zeros((LANES,), jnp.float32) for _ in range(NVREG))
            for ci in range(2):
                s = 2 * bagoff + ci
                drain(s, ci)
                accs = consume(s, ci, accs)
                start_gather(s, bag + GRP, ci)
            store(bag, accs)
        return 0

    lax.fori_loop(0, NSTEPS, step, 0)

    for bag, rbag in ((123, 126), (124, 127), (125, None), (126, None),
                      (127, None)):
        s0 = (2 * bag) % NBUF
        accs = tuple(jnp.zeros((LANES,), jnp.float32) for _ in range(NVREG))
        for ci in range(2):
            drain(s0 + ci, ci)
            accs = consume(s0 + ci, ci, accs)
            if rbag is not None:
                start_gather(s0 + ci, rbag, ci)
        store(bag, accs)

    pltpu.sync_copy(out_v, out_hbm.at[pl.ds(wid * BPW, BPW)])


@jax.jit
def _bow(indices, table):
    mesh = plsc.VectorSubcoreMesh(core_axis_name="c", subcore_axis_name="s")
    return pl.kernel(
        _bow_body,
        mesh=mesh,
        compiler_params=pltpu.CompilerParams(
            needs_layout_passes=False, use_tc_tiling_on_sc=False),
        out_type=jax.ShapeDtypeStruct((B, D), jnp.float32),
        scratch_types=[
            pltpu.VMEM((BPW, L), jnp.int32),
            pltpu.VMEM((NBUF, CHMAX, D), jnp.float32),
            pltpu.VMEM((BPW, D), jnp.float32),
            pltpu.SemaphoreType.DMA,
            pltpu.SemaphoreType.DMA,
            pltpu.SemaphoreType.DMA,
            pltpu.SemaphoreType.DMA,
            pltpu.SemaphoreType.DMA,
            pltpu.SemaphoreType.DMA,
        ],
    )(indices, table)


def kernel(indices, table):
    return _bow(indices, table)


# speedup vs baseline: 1.0181x; 1.0181x over previous
import functools

import jax
import jax.numpy as jnp
from jax import lax
from jax.experimental import pallas as pl
from jax.experimental.pallas import tpu as pltpu
from jax.experimental.pallas import tpu_sc as plsc

B = 4096
L = 200
V = 100000
D = 128

NC = 2
NS = 16
LANES = 16
NW = NC * NS
BPW = B // NW
CHUNKS = ((0, 104), (104, 96))
CHMAX = 104
NBUF = 6
NVREG = D // LANES


def _bow_body(idx_hbm, table_hbm, out_hbm, idx_v, buf_v, out_v,
              sem0, sem1, sem2, sem3, sem4, sem5):
    wid = lax.axis_index("s") * NC + lax.axis_index("c")
    sems = (sem0, sem1, sem2, sem3, sem4, sem5)
    inv = jnp.full((LANES,), 1.0 / L, dtype=jnp.float32)

    pltpu.sync_copy(idx_hbm.at[pl.ds(wid * BPW, BPW)], idx_v)

    def start_gather(slot, bag, ci):
        off, ch = CHUNKS[ci]
        pltpu.make_async_copy(
            table_hbm.at[idx_v.at[bag, pl.ds(off, ch)]],
            buf_v.at[slot, pl.ds(0, ch)],
            sems[slot],
        ).start()

    def drain(slot, ci):
        off, ch = CHUNKS[ci]
        pltpu.make_async_copy(
            table_hbm.at[idx_v.at[0, pl.ds(off, ch)]],
            buf_v.at[slot, pl.ds(0, ch)],
            sems[slot],
        ).wait()

    UNROLL = 2

    def consume(slot, ci, accs):
        _, ch = CHUNKS[ci]

        def row_add(i, accs):
            l = i * UNROLL
            out = []
            for k in range(NVREG):
                a = accs[k]
                parts = [
                    buf_v[slot, l + u, pl.ds(k * LANES, LANES)]
                    for u in range(UNROLL)
                ]
                while len(parts) > 1:
                    parts = [
                        parts[j] + parts[j + 1]
                        for j in range(0, len(parts), 2)
                    ]
                out.append(a + parts[0])
            return tuple(out)

        return lax.fori_loop(0, ch // UNROLL, row_add, accs)

    def store(bag, accs):
        for k in range(NVREG):
            out_v[bag, pl.ds(k * LANES, LANES)] = accs[k] * inv

    GRP = NBUF // 2
    NSTEPS = 41

    for bagoff in range(GRP):
        for ci in range(2):
            start_gather(2 * bagoff + ci, bagoff, ci)

    def step(i, _):
        for bagoff in range(GRP):
            bag = GRP * i + bagoff
            accs = tuple(
                jnp.zeros((LANES,), jnp.float32) for _ in range(NVREG))
            for ci in range(2):
                s = 2 * bagoff + ci
                drain(s, ci)
                accs = consume(s, ci, accs)
                start_gather(s, bag + GRP, ci)
            store(bag, accs)
        return 0

    lax.fori_loop(0, NSTEPS, step, 0)

    for bag, rbag in ((123, 126), (124, 127), (125, None), (126, None),
                      (127, None)):
        s0 = (2 * bag) % NBUF
        accs = tuple(jnp.zeros((LANES,), jnp.float32) for _ in range(NVREG))
        for ci in range(2):
            drain(s0 + ci, ci)
            accs = consume(s0 + ci, ci, accs)
            if rbag is not None:
                start_gather(s0 + ci, rbag, ci)
        store(bag, accs)

    pltpu.sync_copy(out_v, out_hbm.at[pl.ds(wid * BPW, BPW)])


@jax.jit
def _bow(indices, table):
    mesh = plsc.VectorSubcoreMesh(core_axis_name="c", subcore_axis_name="s")
    return pl.kernel(
        _bow_body,
        mesh=mesh,
        compiler_params=pltpu.CompilerParams(
            needs_layout_passes=False, use_tc_tiling_on_sc=False),
        out_type=jax.ShapeDtypeStruct((B, D), jnp.float32),
        scratch_types=[
            pltpu.VMEM((BPW, L), jnp.int32),
            pltpu.VMEM((NBUF, CHMAX, D), jnp.float32),
            pltpu.VMEM((BPW, D), jnp.float32),
            pltpu.SemaphoreType.DMA,
            pltpu.SemaphoreType.DMA,
            pltpu.SemaphoreType.DMA,
            pltpu.SemaphoreType.DMA,
            pltpu.SemaphoreType.DMA,
            pltpu.SemaphoreType.DMA,
        ],
    )(indices, table)


def kernel(indices, table):
    return _bow(indices, table)
